# baseline (speedup 1.0000x reference)
import jax
import jax.numpy as jnp
from jax.experimental import pallas as pl
from jax.experimental.pallas import tpu as pltpu

_N, _P, _D = 4, 4096, 3
_QC = 1024
_NQ = _P // _QC


def _chamfer_kernel(src_ref, tgt_ref, out_ref, min_ref):
    b = pl.program_id(0)
    j = pl.program_id(1)

    St = src_ref[...]
    T = tgt_ref[0]

    y2 = jnp.sum(T * T, axis=1, keepdims=True)
    y2_hi = y2.astype(jnp.bfloat16).astype(jnp.float32)
    y2_lo = y2 - y2_hi
    L = jnp.concatenate([T, y2_hi, y2_lo], axis=1)
    ones_p = jnp.ones((1, _P), jnp.float32)
    R = jnp.concatenate([-2.0 * St, ones_p, ones_p],
                        axis=0)

    d = jax.lax.dot_general(
        L, R, (((1,), (0,)), ((), ())),
        preferred_element_type=jnp.float32,
    )
    m = jnp.min(d, axis=0, keepdims=True)

    @pl.when(j == 0)
    def _():
        min_ref[...] = m

    @pl.when(j > 0)
    def _():
        min_ref[...] = jnp.minimum(min_ref[...], m)

    @pl.when(jnp.logical_and(b == 0, j == 0))
    def _():
        out_ref[...] = jnp.zeros_like(out_ref)

    @pl.when(j == _NQ - 1)
    def _():
        x2 = jnp.sum(St * St, axis=0, keepdims=True)
        s = jnp.sum(min_ref[...] + x2, keepdims=True) * (1.0 / _N)
        out_ref[...] += s


def kernel(source_cloud, target_cloud):
    src_t = source_cloud.reshape(_N * _P, _D).T
    out = pl.pallas_call(
        _chamfer_kernel,
        grid=(_N, _NQ),
        in_specs=[
            pl.BlockSpec((_D, _P), lambda b, j: (0, b)),
            pl.BlockSpec((1, _QC, _D), lambda b, j: (b, j, 0)),
        ],
        out_specs=pl.BlockSpec((1, 1), lambda b, j: (0, 0)),
        out_shape=jax.ShapeDtypeStruct((1, 1), jnp.float32),
        scratch_shapes=[pltpu.VMEM((1, _P), jnp.float32)],
    )(src_t, target_cloud)
    return out[0, 0]
